# baseline (speedup 1.0000x reference)
import dataclasses
import functools

import jax
import jax.numpy as jnp
import numpy as np
from jax import lax
from jax.experimental import pallas as pl
from jax.experimental.pallas import tpu as pltpu
from jax.experimental.pallas import tpu_sc as plsc

_N = 10000
_E = 320000
_C = 128
_LH = 128
_NCORE = 2
_NSUB = 16
_NW = _NCORE * _NSUB
_CHUNK = 96
_NCH0 = 120
_NCH1 = 90
_NCHT = _NSUB * (_NCH0 + _NCH1)
_EPAD = _NCHT * _CHUNK
_NPAD = 10112
_RPT = _NPAD // _NSUB
_ZCOPIES = _RPT // _CHUNK
_ZTAIL = _RPT - _ZCOPIES * _CHUNK

_PERM = np.concatenate(
    [np.concatenate([np.arange(32 * g, 32 * g + 32, 2),
                     np.arange(32 * g + 1, 32 * g + 32, 2)])
     for g in range(_C // 32)])

_RB = 2000


def _sc_segment_sum(m, comb_t):
  mesh = plsc.VectorSubcoreMesh(core_axis_name="c", subcore_axis_name="s")
  cp = pltpu.CompilerParams()
  if "needs_layout_passes" in pltpu.CompilerParams.__dataclass_fields__:
    cp = dataclasses.replace(cp, needs_layout_passes=False)
  if "use_tc_tiling_on_sc" in pltpu.CompilerParams.__dataclass_fields__:
    cp = dataclasses.replace(cp, use_tc_tiling_on_sc=False)

  @functools.partial(
      pl.kernel,
      out_type=jax.ShapeDtypeStruct((_NCORE, _NPAD, _C), jnp.float32),
      mesh=mesh,
      compiler_params=cp,
      scratch_types=(
          [pltpu.VMEM((_CHUNK, _C // 2), jnp.int32)] * 3
          + [pltpu.VMEM((_CHUNK, _C), jnp.float32)] * 2
          + [pltpu.VMEM((3, _CHUNK), jnp.int32)] * 6
          + [pltpu.VMEM_SHARED((_NPAD, _C), jnp.float32)]
          + [pltpu.SemaphoreType.DMA] * 11
      ),
  )
  def seg(m_hbm, comb_hbm, out_hbm, *sc):
    rbf = list(sc[0:3])
    rf32 = list(sc[3:5])
    idx = list(sc[5:11])
    acc = sc[11]
    gsem = list(sc[12:15])
    ssem = list(sc[15:17])
    isem = list(sc[17:23])

    c = lax.axis_index("c")
    s = lax.axis_index("s")
    nch = jnp.where(c == 0, _NCH0, _NCH1)
    base = jnp.where(c == 0, s * _NCH0, _NSUB * _NCH0 + s * _NCH1)

    @pl.loop(0, _CHUNK)
    def _fill_zero(r):
      for v in range(_C // 16):
        rf32[0][r, pl.ds(v * 16, 16)] = jnp.zeros((16,), jnp.float32)

    @pl.loop(0, _ZCOPIES)
    def _zero_acc(b):
      pltpu.sync_copy(rf32[0], acc.at[pl.ds(s * _RPT + b * _CHUNK, _CHUNK)])

    pltpu.sync_copy(rf32[0].at[pl.ds(0, _ZTAIL)],
                    acc.at[pl.ds(s * _RPT + _ZCOPIES * _CHUNK, _ZTAIL)])

    plsc.subcore_barrier()

    def scale_rows(rbf_v, rf_v, idx_v):
      @pl.loop(0, _CHUNK, step=16)
      def _scale(e0):
        wv = plsc.bitcast(idx_v[2, pl.ds(e0, 16)], jnp.float32)
        for t in range(16):
          w = wv[t]
          for g in range(_C // 32):
            ab = plsc.bitcast(rbf_v[e0 + t, pl.ds(16 * g, 16)], jnp.bfloat16)
            a, b = plsc.unpack(ab, format=plsc.PackFormat.INTERLEAVED)
            rf_v[e0 + t, pl.ds(32 * g, 16)] = a * w
            rf_v[e0 + t, pl.ds(32 * g + 16, 16)] = b * w

    for k in range(3):
      pltpu.sync_copy(comb_hbm.at[base + k], idx[k])
    for k in range(2):
      pltpu.async_copy(m_hbm.at[idx[k].at[0]], rbf[k], gsem[k])

    @pl.loop(0, _NCH0, step=6)
    def _edges(j):
      for k_off in range(6):
        b = k_off % 3
        b2 = (k_off + 2) % 3
        fb = k_off % 2
        i_cur = k_off
        i_2 = (k_off + 2) % 6
        i_3 = (k_off + 3) % 6
        k = j + k_off

        @pl.when(k < nch)
        def _blk(k=k, b=b, b2=b2, fb=fb, i_cur=i_cur, i_2=i_2, i_3=i_3):
          pltpu.make_async_copy(m_hbm.at[idx[i_cur].at[0]], rbf[b],
                                gsem[b]).wait()

          @pl.when(k >= 2)
          def _wait_scat():
            pltpu.make_async_copy(rf32[fb], acc.at[idx[i_cur].at[1]],
                                  ssem[fb]).wait()

          scale_rows(rbf[b], rf32[fb], idx[i_cur])
          pltpu.async_copy(rf32[fb], acc.at[idx[i_cur].at[1]], ssem[fb],
                           add=True)

          @pl.when(k + 2 < nch)
          def _nxt():
            @pl.when(k >= 1)
            def _wait_idx():
              pltpu.make_async_copy(comb_hbm.at[base + k + 2], idx[i_2],
                                    isem[i_2]).wait()

            pltpu.async_copy(m_hbm.at[idx[i_2].at[0]], rbf[b2], gsem[b2])

          @pl.when(k + 3 < nch)
          def _pref():
            pltpu.async_copy(comb_hbm.at[base + k + 3], idx[i_3], isem[i_3])

    for t in range(2):
      pltpu.make_async_copy(rf32[t], acc.at[idx[t].at[1]], ssem[t]).wait()

    plsc.subcore_barrier()
    pltpu.sync_copy(acc.at[pl.ds(s * _RPT, _RPT)],
                    out_hbm.at[c, pl.ds(s * _RPT, _RPT)])

  return seg(m, comb_t)


def _tc_matmul(x, w):
  def body(x_ref, w_ref, o_ref):
    o_ref[...] = lax.dot_general(
        x_ref[...], w_ref[...], (((1,), (0,)), ((), ())),
        preferred_element_type=jnp.float32).astype(jnp.bfloat16)

  return pl.pallas_call(
      body,
      grid=(_N // _RB,),
      in_specs=[pl.BlockSpec((_RB, _C), lambda i: (i, 0)),
                pl.BlockSpec((_C, _C), lambda i: (0, 0))],
      out_specs=pl.BlockSpec((_RB, _C), lambda i: (i, 0)),
      out_shape=jax.ShapeDtypeStruct((_N, _C), jnp.bfloat16),
  )(x, w)


def _gru_block(p_ref, h, wih_ref, whh_ref, bih_ref, bhh_ref):
  agg = p_ref[0] + p_ref[1]
  gi = lax.dot_general(agg, wih_ref[...], (((1,), (1,)), ((), ())),
                       preferred_element_type=jnp.float32) + bih_ref[...]
  gh = lax.dot_general(h, whh_ref[...], (((1,), (1,)), ((), ())),
                       preferred_element_type=jnp.float32) + bhh_ref[...]
  r = jax.nn.sigmoid(gi[:, :_C] + gh[:, :_C])
  z = jax.nn.sigmoid(gi[:, _C:2 * _C] + gh[:, _C:2 * _C])
  n = jnp.tanh(gi[:, 2 * _C:] + r * gh[:, 2 * _C:])
  return (1.0 - z) * n + z * h


def _tc_gru_next(p, h, w_ih, w_hh, b_ih2, b_hh2, w_next):
  def body(p_ref, h_ref, wih_ref, whh_ref, bih_ref, bhh_ref, wn_ref,
           x_ref, m_ref):
    x = _gru_block(p_ref, h_ref[...], wih_ref, whh_ref, bih_ref, bhh_ref)
    x_ref[...] = x
    m_ref[...] = lax.dot_general(
        x, wn_ref[...], (((1,), (0,)), ((), ())),
        preferred_element_type=jnp.float32).astype(jnp.bfloat16)

  return pl.pallas_call(
      body,
      grid=(_N // _RB,),
      in_specs=[
          pl.BlockSpec((_NCORE, _RB, _C), lambda i: (0, i, 0)),
          pl.BlockSpec((_RB, _C), lambda i: (i, 0)),
          pl.BlockSpec((3 * _C, _C), lambda i: (0, 0)),
          pl.BlockSpec((3 * _C, _C), lambda i: (0, 0)),
          pl.BlockSpec((1, 3 * _C), lambda i: (0, 0)),
          pl.BlockSpec((1, 3 * _C), lambda i: (0, 0)),
          pl.BlockSpec((_C, _C), lambda i: (0, 0)),
      ],
      out_specs=[pl.BlockSpec((_RB, _C), lambda i: (i, 0)),
                 pl.BlockSpec((_RB, _C), lambda i: (i, 0))],
      out_shape=[jax.ShapeDtypeStruct((_N, _C), jnp.float32),
                 jax.ShapeDtypeStruct((_N, _C), jnp.bfloat16)],
  )(p, h, w_ih, w_hh, b_ih2, b_hh2, w_next)


def _tc_gru_lstm(p, h, w_ih, w_hh, b_ih2, b_hh2, lstm_w_ih, lb2):
  def body(p_ref, h_ref, wih_ref, whh_ref, bih_ref, bhh_ref, wl_ref, lb_ref,
           ht_ref, hn_ref, cn_ref):
    x = _gru_block(p_ref, h_ref[...], wih_ref, whh_ref, bih_ref, bhh_ref)
    gates = lax.dot_general(x, wl_ref[...], (((1,), (1,)), ((), ())),
                            preferred_element_type=jnp.float32) + lb_ref[...]
    ii = jax.nn.sigmoid(gates[:, :_LH])
    gg = jnp.tanh(gates[:, 2 * _LH:3 * _LH])
    oo = jax.nn.sigmoid(gates[:, 3 * _LH:])
    cn = ii * gg
    ht_ref[...] = x
    hn_ref[...] = oo * jnp.tanh(cn)
    cn_ref[...] = cn

  return pl.pallas_call(
      body,
      grid=(_N // _RB,),
      in_specs=[
          pl.BlockSpec((_NCORE, _RB, _C), lambda i: (0, i, 0)),
          pl.BlockSpec((_RB, _C), lambda i: (i, 0)),
          pl.BlockSpec((3 * _C, _C), lambda i: (0, 0)),
          pl.BlockSpec((3 * _C, _C), lambda i: (0, 0)),
          pl.BlockSpec((1, 3 * _C), lambda i: (0, 0)),
          pl.BlockSpec((1, 3 * _C), lambda i: (0, 0)),
          pl.BlockSpec((4 * _LH, _C), lambda i: (0, 0)),
          pl.BlockSpec((1, 4 * _LH), lambda i: (0, 0)),
      ],
      out_specs=[pl.BlockSpec((_RB, _C), lambda i: (i, 0)),
                 pl.BlockSpec((_RB, _LH), lambda i: (i, 0)),
                 pl.BlockSpec((_RB, _LH), lambda i: (i, 0))],
      out_shape=[jax.ShapeDtypeStruct((_N, _C), jnp.float32),
                 jax.ShapeDtypeStruct((_N, _LH), jnp.float32),
                 jax.ShapeDtypeStruct((_N, _LH), jnp.float32)],
  )(p, h, w_ih, w_hh, b_ih2, b_hh2, lstm_w_ih, lb2)


def kernel(X, edge_index, edge_weight, ggc_weight, gru_w_ih, gru_w_hh,
           gru_b_ih, gru_b_hh, lstm_w_ih, lstm_w_hh, lstm_b_ih, lstm_b_hh):
  pad = _EPAD - _E
  src_t = jnp.concatenate(
      [edge_index[0], jnp.zeros((pad,), jnp.int32)]).reshape(_NCHT, _CHUNK)
  dst_t = jnp.concatenate(
      [edge_index[1], jnp.zeros((pad,), jnp.int32)]).reshape(_NCHT, _CHUNK)
  ew_t = lax.bitcast_convert_type(
      jnp.concatenate([edge_weight, jnp.zeros((pad,), jnp.float32)]),
      jnp.int32).reshape(_NCHT, _CHUNK)
  comb_t = jnp.stack([src_t, dst_t, ew_t], axis=1)
  bih2 = gru_b_ih.reshape(1, 3 * _C)
  bhh2 = gru_b_hh.reshape(1, 3 * _C)
  lb2 = (lstm_b_ih + lstm_b_hh).reshape(1, 4 * _LH)
  wih_p = gru_w_ih[:, _PERM]

  def as_i32_pairs(mb):
    return lax.bitcast_convert_type(mb.reshape(_N, _C // 2, 2), jnp.int32)

  x = X
  m = _tc_matmul(x, ggc_weight[0])
  for i in range(3):
    p = _sc_segment_sum(as_i32_pairs(m), comb_t)
    if i < 2:
      x, m = _tc_gru_next(p, x, wih_p, gru_w_hh, bih2, bhh2,
                          ggc_weight[i + 1])
    else:
      h_tilde, h_new, c_new = _tc_gru_lstm(p, x, wih_p, gru_w_hh, bih2,
                                           bhh2, lstm_w_ih, lb2)
  return (h_tilde, h_new, c_new)


# speedup vs baseline: 1.6040x; 1.6040x over previous
import dataclasses
import functools

import jax
import jax.numpy as jnp
from jax import lax
from jax.experimental import pallas as pl
from jax.experimental.pallas import tpu as pltpu
from jax.experimental.pallas import tpu_sc as plsc

_N = 10000
_E = 320000
_C = 128
_LH = 128
_NCORE = 2
_NSUB = 16
_NW = _NCORE * _NSUB
_CHUNK = 112
_NCH0 = 138
_NCH1 = 42
_NCHT = _NSUB * (_NCH0 + _NCH1)
_EPAD = _NCHT * _CHUNK
_NPAD = 10112
_RPT = _NPAD // _NSUB
_ZTAIL = _RPT - 5 * _CHUNK

_RB = 2000


def _sc_segment_sum(m, comb_t):
  mesh = plsc.VectorSubcoreMesh(core_axis_name="c", subcore_axis_name="s")
  cp = pltpu.CompilerParams()
  if "needs_layout_passes" in pltpu.CompilerParams.__dataclass_fields__:
    cp = dataclasses.replace(cp, needs_layout_passes=False)

  @functools.partial(
      pl.kernel,
      out_type=jax.ShapeDtypeStruct((_NCORE, _NPAD, _C), jnp.float32),
      mesh=mesh,
      compiler_params=cp,
      scratch_types=(
          [pltpu.VMEM((_CHUNK, _C), jnp.float32)] * 3
          + [pltpu.VMEM((3, _CHUNK), jnp.int32)] * 6
          + [pltpu.VMEM_SHARED((_NPAD, _C), jnp.float32)]
          + [pltpu.SemaphoreType.DMA] * 12
      ),
  )
  def seg(m_hbm, comb_hbm, out_hbm, *sc):
    rows = list(sc[0:3])
    idx = list(sc[3:9])
    acc = sc[9]
    gsem = list(sc[10:13])
    ssem = list(sc[13:16])
    isem = list(sc[16:22])

    c = lax.axis_index("c")
    s = lax.axis_index("s")
    nch = jnp.where(c == 0, _NCH0, _NCH1)
    base = jnp.where(c == 0, s * _NCH0, _NSUB * _NCH0 + s * _NCH1)

    @pl.loop(0, _CHUNK)
    def _fill_zero(r):
      for v in range(_C // 16):
        rows[0][r, pl.ds(v * 16, 16)] = jnp.zeros((16,), jnp.float32)

    @pl.loop(0, _RPT // _CHUNK)
    def _zero_acc(b):
      pltpu.sync_copy(rows[0], acc.at[pl.ds(s * _RPT + b * _CHUNK, _CHUNK)])

    pltpu.sync_copy(rows[0].at[pl.ds(0, _ZTAIL)],
                    acc.at[pl.ds(s * _RPT + 5 * _CHUNK, _ZTAIL)])

    plsc.subcore_barrier()

    def scale_rows(rows_v, idx_v):
      @pl.loop(0, _CHUNK, step=16)
      def _scale(e0):
        wv = plsc.bitcast(idx_v[2, pl.ds(e0, 16)], jnp.float32)
        for k in range(16):
          w = wv[k]
          for v in range(_C // 16):
            sl = (e0 + k, pl.ds(v * 16, 16))
            rows_v[sl] = rows_v[sl] * w

    for k in range(3):
      pltpu.sync_copy(comb_hbm.at[base + k], idx[k])
    for k in range(2):
      pltpu.async_copy(m_hbm.at[idx[k].at[0]], rows[k], gsem[k])

    @pl.loop(0, _NCH0, step=6)
    def _edges(j):
      for k_off in range(6):
        b = k_off % 3
        b2 = (k_off + 2) % 3
        i_cur = k_off
        i_2 = (k_off + 2) % 6
        i_3 = (k_off + 3) % 6
        k = j + k_off

        @pl.when(k < nch)
        def _blk(k=k, b=b, b2=b2, i_cur=i_cur, i_2=i_2, i_3=i_3,
                 k_off=k_off):
          pltpu.make_async_copy(m_hbm.at[idx[i_cur].at[0]], rows[b],
                                gsem[b]).wait()
          scale_rows(rows[b], idx[i_cur])
          pltpu.async_copy(rows[b], acc.at[idx[i_cur].at[1]], ssem[b],
                           add=True)

          @pl.when(k + 2 < nch)
          def _nxt():
            @pl.when(k >= 1)
            def _wait_idx():
              pltpu.make_async_copy(comb_hbm.at[base + k + 2], idx[i_2],
                                    isem[i_2]).wait()

            @pl.when(k >= 1)
            def _wait_scat():
              pltpu.make_async_copy(rows[b2], acc.at[idx[i_2].at[1]],
                                    ssem[b2]).wait()

            pltpu.async_copy(m_hbm.at[idx[i_2].at[0]], rows[b2], gsem[b2])

          @pl.when(k + 3 < nch)
          def _pref():
            pltpu.async_copy(comb_hbm.at[base + k + 3], idx[i_3], isem[i_3])

    for t in range(3):
      pltpu.make_async_copy(rows[t], acc.at[idx[t].at[1]], ssem[t]).wait()

    plsc.subcore_barrier()
    pltpu.sync_copy(acc.at[pl.ds(s * _RPT, _RPT)],
                    out_hbm.at[c, pl.ds(s * _RPT, _RPT)])

  return seg(m, comb_t)


def _tc_matmul(x, w):
  def body(x_ref, w_ref, o_ref):
    o_ref[...] = lax.dot_general(
        x_ref[...], w_ref[...], (((1,), (0,)), ((), ())),
        preferred_element_type=jnp.float32)

  return pl.pallas_call(
      body,
      grid=(_N // _RB,),
      in_specs=[pl.BlockSpec((_RB, _C), lambda i: (i, 0)),
                pl.BlockSpec((_C, _C), lambda i: (0, 0))],
      out_specs=pl.BlockSpec((_RB, _C), lambda i: (i, 0)),
      out_shape=jax.ShapeDtypeStruct((_N, _C), jnp.float32),
  )(x, w)


def _gru_block(p_ref, h, wih_ref, whh_ref, bih_ref, bhh_ref):
  agg = p_ref[0] + p_ref[1]
  gi = lax.dot_general(agg, wih_ref[...], (((1,), (1,)), ((), ())),
                       preferred_element_type=jnp.float32) + bih_ref[...]
  gh = lax.dot_general(h, whh_ref[...], (((1,), (1,)), ((), ())),
                       preferred_element_type=jnp.float32) + bhh_ref[...]
  r = jax.nn.sigmoid(gi[:, :_C] + gh[:, :_C])
  z = jax.nn.sigmoid(gi[:, _C:2 * _C] + gh[:, _C:2 * _C])
  n = jnp.tanh(gi[:, 2 * _C:] + r * gh[:, 2 * _C:])
  return (1.0 - z) * n + z * h


def _tc_gru_next(p, h, w_ih, w_hh, b_ih2, b_hh2, w_next):
  def body(p_ref, h_ref, wih_ref, whh_ref, bih_ref, bhh_ref, wn_ref,
           x_ref, m_ref):
    x = _gru_block(p_ref, h_ref[...], wih_ref, whh_ref, bih_ref, bhh_ref)
    x_ref[...] = x
    m_ref[...] = lax.dot_general(x, wn_ref[...], (((1,), (0,)), ((), ())),
                                 preferred_element_type=jnp.float32)

  return pl.pallas_call(
      body,
      grid=(_N // _RB,),
      in_specs=[
          pl.BlockSpec((_NCORE, _RB, _C), lambda i: (0, i, 0)),
          pl.BlockSpec((_RB, _C), lambda i: (i, 0)),
          pl.BlockSpec((3 * _C, _C), lambda i: (0, 0)),
          pl.BlockSpec((3 * _C, _C), lambda i: (0, 0)),
          pl.BlockSpec((1, 3 * _C), lambda i: (0, 0)),
          pl.BlockSpec((1, 3 * _C), lambda i: (0, 0)),
          pl.BlockSpec((_C, _C), lambda i: (0, 0)),
      ],
      out_specs=[pl.BlockSpec((_RB, _C), lambda i: (i, 0)),
                 pl.BlockSpec((_RB, _C), lambda i: (i, 0))],
      out_shape=[jax.ShapeDtypeStruct((_N, _C), jnp.float32),
                 jax.ShapeDtypeStruct((_N, _C), jnp.float32)],
  )(p, h, w_ih, w_hh, b_ih2, b_hh2, w_next)


def _tc_gru_lstm(p, h, w_ih, w_hh, b_ih2, b_hh2, lstm_w_ih, lb2):
  def body(p_ref, h_ref, wih_ref, whh_ref, bih_ref, bhh_ref, wl_ref, lb_ref,
           ht_ref, hn_ref, cn_ref):
    x = _gru_block(p_ref, h_ref[...], wih_ref, whh_ref, bih_ref, bhh_ref)
    gates = lax.dot_general(x, wl_ref[...], (((1,), (1,)), ((), ())),
                            preferred_element_type=jnp.float32) + lb_ref[...]
    ii = jax.nn.sigmoid(gates[:, :_LH])
    gg = jnp.tanh(gates[:, 2 * _LH:3 * _LH])
    oo = jax.nn.sigmoid(gates[:, 3 * _LH:])
    cn = ii * gg
    ht_ref[...] = x
    hn_ref[...] = oo * jnp.tanh(cn)
    cn_ref[...] = cn

  return pl.pallas_call(
      body,
      grid=(_N // _RB,),
      in_specs=[
          pl.BlockSpec((_NCORE, _RB, _C), lambda i: (0, i, 0)),
          pl.BlockSpec((_RB, _C), lambda i: (i, 0)),
          pl.BlockSpec((3 * _C, _C), lambda i: (0, 0)),
          pl.BlockSpec((3 * _C, _C), lambda i: (0, 0)),
          pl.BlockSpec((1, 3 * _C), lambda i: (0, 0)),
          pl.BlockSpec((1, 3 * _C), lambda i: (0, 0)),
          pl.BlockSpec((4 * _LH, _C), lambda i: (0, 0)),
          pl.BlockSpec((1, 4 * _LH), lambda i: (0, 0)),
      ],
      out_specs=[pl.BlockSpec((_RB, _C), lambda i: (i, 0)),
                 pl.BlockSpec((_RB, _LH), lambda i: (i, 0)),
                 pl.BlockSpec((_RB, _LH), lambda i: (i, 0))],
      out_shape=[jax.ShapeDtypeStruct((_N, _C), jnp.float32),
                 jax.ShapeDtypeStruct((_N, _LH), jnp.float32),
                 jax.ShapeDtypeStruct((_N, _LH), jnp.float32)],
  )(p, h, w_ih, w_hh, b_ih2, b_hh2, lstm_w_ih, lb2)


def kernel(X, edge_index, edge_weight, ggc_weight, gru_w_ih, gru_w_hh,
           gru_b_ih, gru_b_hh, lstm_w_ih, lstm_w_hh, lstm_b_ih, lstm_b_hh):
  pad = _EPAD - _E
  src_t = jnp.concatenate(
      [edge_index[0], jnp.zeros((pad,), jnp.int32)]).reshape(_NCHT, _CHUNK)
  dst_t = jnp.concatenate(
      [edge_index[1], jnp.zeros((pad,), jnp.int32)]).reshape(_NCHT, _CHUNK)
  ew_t = lax.bitcast_convert_type(
      jnp.concatenate([edge_weight, jnp.zeros((pad,), jnp.float32)]),
      jnp.int32).reshape(_NCHT, _CHUNK)
  comb_t = jnp.stack([src_t, dst_t, ew_t], axis=1)
  bih2 = gru_b_ih.reshape(1, 3 * _C)
  bhh2 = gru_b_hh.reshape(1, 3 * _C)
  lb2 = (lstm_b_ih + lstm_b_hh).reshape(1, 4 * _LH)

  x = X
  m = _tc_matmul(x, ggc_weight[0])
  for i in range(3):
    p = _sc_segment_sum(m, comb_t)
    if i < 2:
      x, m = _tc_gru_next(p, x, gru_w_ih, gru_w_hh, bih2, bhh2,
                          ggc_weight[i + 1])
    else:
      h_tilde, h_new, c_new = _tc_gru_lstm(p, x, gru_w_ih, gru_w_hh, bih2,
                                           bhh2, lstm_w_ih, lb2)
  return (h_tilde, h_new, c_new)
